# baseline (speedup 1.0000x reference)
import functools

import jax
import jax.numpy as jnp
from jax import lax
from jax.experimental import pallas as pl
from jax.experimental.pallas import tpu as pltpu
from jax.experimental.pallas import tpu_sc as plsc

N = 10000
E = 320000
IN_DIM = 128
HID = 64

NC = 2
NS = 16
CHUNK = 128
TCH = 160
TH = HID // 2
EPT = TCH * CHUNK
E_PAD = NS * EPT
NP = 10240
RPT = NP // NS
DEGW = 16
SROWS = 128
NZCP = RPT // SROWS

_mesh = plsc.VectorSubcoreMesh(core_axis_name="c", subcore_axis_name="s",
                               num_cores=1)
_mesh2 = plsc.VectorSubcoreMesh(core_axis_name="c", subcore_axis_name="s",
                                num_cores=2)
_sc_params = pltpu.CompilerParams(use_tc_tiling_on_sc=False)


def _zero_stage(stage_v, nrows, width):
    def body(i, c):
        for j in range(width // 16):
            stage_v[i, pl.ds(j * 16, 16)] = jnp.zeros((16,), jnp.float32)
        return c
    lax.fori_loop(0, nrows, body, 0)


@functools.partial(
    pl.kernel,
    out_type=jax.ShapeDtypeStruct((NP, DEGW), jnp.float32),
    mesh=_mesh,
    scratch_types=[
        pltpu.VMEM((TCH, CHUNK), jnp.int32),
        pltpu.VMEM((CHUNK, DEGW), jnp.float32),
        pltpu.VMEM((SROWS, DEGW), jnp.float32),
        pltpu.VMEM_SHARED((NP, DEGW), jnp.float32),
        pltpu.SemaphoreType.DMA,
    ],
    compiler_params=_sc_params,
)
def _deg_kernel(dst_hbm, out_hbm, didx_v, ones_v, stage_v, acc_sh, sem):
    sid = lax.axis_index("s")
    r0 = sid * RPT

    pltpu.async_copy(dst_hbm.at[sid], didx_v, sem)

    def fill_ones(i, c):
        ones_v[i, pl.ds(0, 16)] = jnp.ones((16,), jnp.float32)
        return c
    lax.fori_loop(0, CHUNK, fill_ones, 0)
    _zero_stage(stage_v, SROWS, DEGW)

    def zcp(k, c):
        pltpu.sync_copy(stage_v, acc_sh.at[pl.ds(r0 + k * SROWS, SROWS)])
        return c
    lax.fori_loop(0, NZCP, zcp, 0)
    pltpu.make_async_copy(dst_hbm.at[sid], didx_v, sem).wait()
    plsc.subcore_barrier()

    def chunk_body(c, carry):
        pltpu.sync_copy(ones_v, acc_sh.at[didx_v.at[c]], add=True)
        return carry
    lax.fori_loop(0, TCH, chunk_body, 0)
    plsc.subcore_barrier()

    def ocp(k, c):
        pltpu.sync_copy(acc_sh.at[pl.ds(r0 + k * SROWS, SROWS)], stage_v)
        pltpu.sync_copy(stage_v, out_hbm.at[pl.ds(r0 + k * SROWS, SROWS)])
        return c
    lax.fori_loop(0, NZCP, ocp, 0)


NBUF = 8
NOUT = TCH // NBUF


@functools.partial(
    pl.kernel,
    out_type=jax.ShapeDtypeStruct((NC, NP, TH), jnp.float32),
    mesh=_mesh2,
    scratch_types=[
        pltpu.VMEM((TCH, CHUNK), jnp.int32),
        pltpu.VMEM((TCH, CHUNK), jnp.int32),
        pltpu.VMEM((NBUF, CHUNK, TH), jnp.float32),
        pltpu.VMEM((SROWS, TH), jnp.float32),
        pltpu.VMEM_SHARED((NP, TH), jnp.float32),
        pltpu.VMEM_SHARED((NP, TH), jnp.float32),
        pltpu.SemaphoreType.DMA((NBUF,)),
        pltpu.SemaphoreType.DMA,
    ],
    compiler_params=_sc_params,
    cost_estimate=pl.CostEstimate(
        flops=E_PAD * TH,
        bytes_accessed=2 * NC * E_PAD * TH * 4,
        transcendentals=0),
)
def _prop_kernel(t_hbm, src_hbm, dst_hbm, out_hbm,
                 sidx_v, didx_v, rows_v, stage_v, t_sh, acc_sh, gsem, csem):
    cid = lax.axis_index("c")
    sid = lax.axis_index("s")
    r0 = sid * RPT

    pltpu.async_copy(src_hbm.at[sid], sidx_v, csem)
    pltpu.async_copy(dst_hbm.at[sid], didx_v, csem)

    def stg(k, c):
        pltpu.sync_copy(t_hbm.at[cid, pl.ds(r0 + k * SROWS, SROWS)], stage_v)
        pltpu.sync_copy(stage_v, t_sh.at[pl.ds(r0 + k * SROWS, SROWS)])
        return c
    lax.fori_loop(0, NZCP, stg, 0)

    _zero_stage(stage_v, SROWS, TH)

    def zcp(k, c):
        pltpu.sync_copy(stage_v, acc_sh.at[pl.ds(r0 + k * SROWS, SROWS)])
        return c
    lax.fori_loop(0, NZCP, zcp, 0)
    pltpu.make_async_copy(src_hbm.at[sid], sidx_v, csem).wait()
    pltpu.make_async_copy(dst_hbm.at[sid], didx_v, csem).wait()
    plsc.subcore_barrier()

    for b in range(NBUF):
        pltpu.async_copy(t_sh.at[sidx_v.at[b]], rows_v.at[b], gsem.at[b])

    def outer(o, carry):
        for b in range(NBUF):
            c = o * NBUF + b
            pltpu.make_async_copy(t_sh.at[sidx_v.at[c]], rows_v.at[b],
                                  gsem.at[b]).wait()
            pltpu.sync_copy(rows_v.at[b], acc_sh.at[didx_v.at[c]], add=True)

            @pl.when(o < NOUT - 1)
            def _():
                pltpu.async_copy(t_sh.at[sidx_v.at[c + NBUF]], rows_v.at[b],
                                 gsem.at[b])
        return carry
    lax.fori_loop(0, NOUT, outer, 0)
    plsc.subcore_barrier()

    def ocp(k, c):
        pltpu.sync_copy(acc_sh.at[pl.ds(r0 + k * SROWS, SROWS)], stage_v)
        pltpu.sync_copy(stage_v, out_hbm.at[cid, pl.ds(r0 + k * SROWS, SROWS)])
        return c
    lax.fori_loop(0, NZCP, ocp, 0)


RB = 1024
_GRID = NP // RB


def _full(shape):
    return pl.BlockSpec(shape, lambda i: tuple(0 for _ in shape))


def _rows(width, leading=None):
    if leading is None:
        return pl.BlockSpec((RB, width), lambda i: (i, 0))
    return pl.BlockSpec((leading, RB, width), lambda i: (0, i, 0))


def _split_t(t_ref, th):
    t_ref[0] = th[:, :TH]
    t_ref[1] = th[:, TH:]


_T_SHAPE = jax.ShapeDtypeStruct((NC, NP, TH), jnp.float32)


def _u1_body(x_ref, w_ref, u_ref):
    u_ref[...] = jnp.dot(x_ref[...], w_ref[...],
                         preferred_element_type=jnp.float32)


def _stage_u1(x_pad, w1):
    return pl.pallas_call(
        _u1_body,
        grid=(_GRID,),
        in_specs=[_rows(IN_DIM), _full((IN_DIM, HID))],
        out_specs=_rows(HID),
        out_shape=jax.ShapeDtypeStruct((NP, HID), jnp.float32),
    )(x_pad, w1)


def _stage0_body(cnt_ref, u_ref, dinv_ref, t_ref):
    d = lax.rsqrt(cnt_ref[:, 0:1] + 1.0)
    dinv_ref[...] = d
    _split_t(t_ref, d * u_ref[...])


def _stage0(cnt, u1):
    return pl.pallas_call(
        _stage0_body,
        grid=(_GRID,),
        in_specs=[_rows(DEGW), _rows(HID)],
        out_specs=[_rows(1), _rows(TH, leading=NC)],
        out_shape=[
            jax.ShapeDtypeStruct((NP, 1), jnp.float32),
            _T_SHAPE,
        ],
    )(cnt, u1)


def _combine(S_ref, uprev, d, b, relu):
    S = jnp.concatenate([S_ref[0], S_ref[1]], axis=1)
    g = d * S + (d * d) * uprev + b
    if relu:
        g = jnp.maximum(g, 0.0)
    return g


def _project_body(S_ref, up_ref, d_ref, b_ref, w_ref, u_ref, t_ref, *, relu):
    g = _combine(S_ref, up_ref[...], d_ref[...], b_ref[...], relu)
    u = jnp.dot(g, w_ref[...], preferred_element_type=jnp.float32)
    u_ref[...] = u
    _split_t(t_ref, d_ref[...] * u)


def _stage_project(S, uprev, dinv, b, w, relu):
    return pl.pallas_call(
        functools.partial(_project_body, relu=relu),
        grid=(_GRID,),
        in_specs=[_rows(TH, leading=NC), _rows(HID), _rows(1),
                  _full((1, HID)), _full((HID, HID))],
        out_specs=[_rows(HID), _rows(TH, leading=NC)],
        out_shape=[
            jax.ShapeDtypeStruct((NP, HID), jnp.float32),
            _T_SHAPE,
        ],
    )(S, uprev, dinv, b.reshape(1, HID), w)


def _project2_body(S_ref, up_ref, d_ref, b_ref, wa_ref, ws_ref,
                   ua_ref, ta_ref, us_ref, ts_ref):
    g = _combine(S_ref, up_ref[...], d_ref[...], b_ref[...], relu=False)
    d = d_ref[...]
    ua = jnp.dot(g, wa_ref[...], preferred_element_type=jnp.float32)
    ua_ref[...] = ua
    _split_t(ta_ref, d * ua)
    us = jnp.dot(g, ws_ref[...], preferred_element_type=jnp.float32)
    us_ref[...] = us
    _split_t(ts_ref, d * us)


def _stage_project2(S, uprev, dinv, b, wa, ws):
    return pl.pallas_call(
        _project2_body,
        grid=(_GRID,),
        in_specs=[_rows(TH, leading=NC), _rows(HID), _rows(1),
                  _full((1, HID)), _full((HID, HID)), _full((HID, HID))],
        out_specs=[_rows(HID), _rows(TH, leading=NC),
                   _rows(HID), _rows(TH, leading=NC)],
        out_shape=[jax.ShapeDtypeStruct((NP, HID), jnp.float32), _T_SHAPE,
                   jax.ShapeDtypeStruct((NP, HID), jnp.float32), _T_SHAPE],
    )(S, uprev, dinv, b.reshape(1, HID), wa, ws)


def _stage_relu_body(S_ref, up_ref, d_ref, b_ref, a_ref, t_ref):
    g = _combine(S_ref, up_ref[...], d_ref[...], b_ref[...], relu=True)
    a_ref[...] = g
    _split_t(t_ref, d_ref[...] * g)


def _stage_relu(S, uprev, dinv, b):
    return pl.pallas_call(
        _stage_relu_body,
        grid=(_GRID,),
        in_specs=[_rows(TH, leading=NC), _rows(HID), _rows(1), _full((1, HID))],
        out_specs=[_rows(HID), _rows(TH, leading=NC)],
        out_shape=[
            jax.ShapeDtypeStruct((NP, HID), jnp.float32),
            _T_SHAPE,
        ],
    )(S, uprev, dinv, b.reshape(1, HID))


def _stage_out_body(S_ref, up_ref, d_ref, b_ref, w_ref, x_ref):
    d = d_ref[...]
    S = jnp.concatenate([S_ref[0], S_ref[1]], axis=1)
    v = d * S + (d * d) * up_ref[...]
    x_ref[...] = jnp.dot(v, w_ref[...], preferred_element_type=jnp.float32) + b_ref[...]


def _stage_out(S, uprev, dinv, b, w):
    return pl.pallas_call(
        _stage_out_body,
        grid=(_GRID,),
        in_specs=[_rows(TH, leading=NC), _rows(HID), _rows(1),
                  _full((1, IN_DIM)), _full((HID, IN_DIM))],
        out_specs=_rows(IN_DIM),
        out_shape=jax.ShapeDtypeStruct((N, IN_DIM), jnp.float32),
    )(S, uprev, dinv, b.reshape(1, IN_DIM), w)


def _stage_hs_body(S_ref, up_ref, d_ref, b_ref, hs_ref):
    hs_ref[...] = _combine(S_ref, up_ref[...], d_ref[...], b_ref[...],
                           relu=False)


def _stage_hs(S, uprev, dinv, b):
    return pl.pallas_call(
        _stage_hs_body,
        grid=(_GRID,),
        in_specs=[_rows(TH, leading=NC), _rows(HID), _rows(1), _full((1, HID))],
        out_specs=_rows(HID),
        out_shape=jax.ShapeDtypeStruct((NP, HID), jnp.float32),
    )(S, uprev, dinv, b.reshape(1, HID))


MMB = 1024


def _struct_body(a_ref, b_ref, o_ref):
    o_ref[...] = lax.dot_general(
        a_ref[...], b_ref[...], (((1,), (1,)), ((), ())),
        preferred_element_type=jnp.float32)


def _struct_body2(a_ref, b_ref, sprev_ref, o_ref):
    del sprev_ref
    o_ref[...] = lax.dot_general(
        a_ref[...], b_ref[...], (((1,), (1,)), ((), ())),
        preferred_element_type=jnp.float32)


_G = pl.cdiv(N, MMB)
_GH = _G // 2


def _struct_cost(frac):
    return pl.CostEstimate(
        flops=int(2 * N * N * HID * frac),
        bytes_accessed=int((N * N * 4) * frac) + 2 * N * HID * 4,
        transcendentals=0)


def _struct_decode_top(hs):
    return pl.pallas_call(
        _struct_body,
        grid=(_GH, _G),
        in_specs=[
            pl.BlockSpec((MMB, HID), lambda i, j: (i, 0)),
            pl.BlockSpec((MMB, HID), lambda i, j: (j, 0)),
        ],
        out_specs=pl.BlockSpec((MMB, MMB), lambda i, j: (i, j)),
        out_shape=jax.ShapeDtypeStruct((N, N), jnp.float32),
        cost_estimate=_struct_cost(0.5),
    )(hs, hs)


def _struct_decode_bottom(hs, s_top):
    return pl.pallas_call(
        _struct_body2,
        grid=(_G - _GH, _G),
        in_specs=[
            pl.BlockSpec((MMB, HID), lambda i, j: (i + _GH, 0)),
            pl.BlockSpec((MMB, HID), lambda i, j: (j, 0)),
            pl.BlockSpec(memory_space=pl.ANY),
        ],
        out_specs=pl.BlockSpec((MMB, MMB), lambda i, j: (i + _GH, j)),
        out_shape=jax.ShapeDtypeStruct((N, N), jnp.float32),
        input_output_aliases={2: 0},
        cost_estimate=_struct_cost(0.5),
    )(hs, hs, s_top)


def kernel(x, edge_index, enc_W1, enc_b1, enc_W2, enc_b2,
           attr_W1, attr_b1, attr_W2, attr_b2, str_W1, str_b1):
    src = edge_index[0]
    dst = edge_index[1]
    pad = E_PAD - E
    src_p = jnp.concatenate([src, jnp.full((pad,), N, jnp.int32)])
    dst_p = jnp.concatenate([dst, jnp.full((pad,), N, jnp.int32)])
    src_t = src_p.reshape(NS, TCH, CHUNK)
    dst_t = dst_p.reshape(NS, TCH, CHUNK)
    dst_deg = dst_t

    x_pad = jnp.pad(x, ((0, NP - N), (0, 0)))

    u1 = _stage_u1(x_pad, enc_W1)
    cnt = _deg_kernel(dst_deg)
    dinv, t1 = _stage0(cnt, u1)

    S1 = _prop_kernel(t1, src_t, dst_t)
    u2, t2 = _stage_project(S1, u1, dinv, enc_b1, enc_W2, relu=True)

    S2 = _prop_kernel(t2, src_t, dst_t)
    u3, t3, u4, t4 = _stage_project2(S2, u2, dinv, enc_b2, attr_W1, str_W1)

    S4 = _prop_kernel(t4, src_t, dst_t)
    hs = _stage_hs(S4, u4, dinv, str_b1)

    S3 = _prop_kernel(t3, src_t, dst_t)
    s_top = _struct_decode_top(hs)
    b_attr1 = attr_b1 + 0.0 * s_top[0, 0]
    a, t5 = _stage_relu(S3, u3, dinv, b_attr1)
    S5 = _prop_kernel(t5, src_t, dst_t)
    s_ = _struct_decode_bottom(hs, s_top)
    b_attr2 = attr_b2 + 0.0 * s_[0, 0]
    x_ = _stage_out(S5, a, dinv, b_attr2, attr_W2)
    return x_, s_


# speedup vs baseline: 1.1020x; 1.1020x over previous
import functools

import jax
import jax.numpy as jnp
from jax import lax
from jax.experimental import pallas as pl
from jax.experimental.pallas import tpu as pltpu
from jax.experimental.pallas import tpu_sc as plsc

N = 10000
E = 320000
IN_DIM = 128
HID = 64

NC = 2
NS = 16
CHUNK = 128
TCH = 160
TH = HID // 2
EPT = TCH * CHUNK
E_PAD = NS * EPT
NP = 10240
RPT = NP // NS
DEGW = 16
SROWS = 128
NZCP = RPT // SROWS

_mesh = plsc.VectorSubcoreMesh(core_axis_name="c", subcore_axis_name="s",
                               num_cores=1)
_mesh2 = plsc.VectorSubcoreMesh(core_axis_name="c", subcore_axis_name="s",
                                num_cores=2)
_sc_params = pltpu.CompilerParams(use_tc_tiling_on_sc=False)


def _zero_stage(stage_v, nrows, width):
    def body(i, c):
        for j in range(width // 16):
            stage_v[i, pl.ds(j * 16, 16)] = jnp.zeros((16,), jnp.float32)
        return c
    lax.fori_loop(0, nrows, body, 0)


@functools.partial(
    pl.kernel,
    out_type=jax.ShapeDtypeStruct((NP, DEGW), jnp.float32),
    mesh=_mesh,
    scratch_types=[
        pltpu.VMEM((TCH, CHUNK), jnp.int32),
        pltpu.VMEM((CHUNK, DEGW), jnp.float32),
        pltpu.VMEM((SROWS, DEGW), jnp.float32),
        pltpu.VMEM_SHARED((NP, DEGW), jnp.float32),
        pltpu.SemaphoreType.DMA,
    ],
    compiler_params=_sc_params,
)
def _deg_kernel(dst_hbm, out_hbm, didx_v, ones_v, stage_v, acc_sh, sem):
    sid = lax.axis_index("s")
    r0 = sid * RPT

    pltpu.async_copy(dst_hbm.at[sid], didx_v, sem)

    def fill_ones(i, c):
        ones_v[i, pl.ds(0, 16)] = jnp.ones((16,), jnp.float32)
        return c
    lax.fori_loop(0, CHUNK, fill_ones, 0)
    _zero_stage(stage_v, SROWS, DEGW)

    def zcp(k, c):
        pltpu.sync_copy(stage_v, acc_sh.at[pl.ds(r0 + k * SROWS, SROWS)])
        return c
    lax.fori_loop(0, NZCP, zcp, 0)
    pltpu.make_async_copy(dst_hbm.at[sid], didx_v, sem).wait()
    plsc.subcore_barrier()

    def chunk_body(c, carry):
        pltpu.sync_copy(ones_v, acc_sh.at[didx_v.at[c]], add=True)
        return carry
    lax.fori_loop(0, TCH, chunk_body, 0)
    plsc.subcore_barrier()

    def ocp(k, c):
        pltpu.sync_copy(acc_sh.at[pl.ds(r0 + k * SROWS, SROWS)], stage_v)
        pltpu.sync_copy(stage_v, out_hbm.at[pl.ds(r0 + k * SROWS, SROWS)])
        return c
    lax.fori_loop(0, NZCP, ocp, 0)


NBUF = 8
PREF = 4
NOUT = TCH // NBUF


@functools.partial(
    pl.kernel,
    out_type=jax.ShapeDtypeStruct((NC, NP, TH), jnp.float32),
    mesh=_mesh2,
    scratch_types=[
        pltpu.VMEM((TCH, CHUNK), jnp.int32),
        pltpu.VMEM((TCH, CHUNK), jnp.int32),
        pltpu.VMEM((NBUF, CHUNK, TH), jnp.float32),
        pltpu.VMEM((SROWS, TH), jnp.float32),
        pltpu.VMEM_SHARED((NP, TH), jnp.float32),
        pltpu.VMEM_SHARED((NP, TH), jnp.float32),
        pltpu.SemaphoreType.DMA((NBUF,)),
        pltpu.SemaphoreType.DMA((NBUF,)),
        pltpu.SemaphoreType.DMA,
    ],
    compiler_params=_sc_params,
    cost_estimate=pl.CostEstimate(
        flops=E_PAD * TH,
        bytes_accessed=2 * NC * E_PAD * TH * 4,
        transcendentals=0),
)
def _prop_kernel(t_hbm, src_hbm, dst_hbm, out_hbm,
                 sidx_v, didx_v, rows_v, stage_v, t_sh, acc_sh,
                 gsem, ssem, csem):
    cid = lax.axis_index("c")
    sid = lax.axis_index("s")
    r0 = sid * RPT

    pltpu.async_copy(src_hbm.at[sid], sidx_v, csem)
    pltpu.async_copy(dst_hbm.at[sid], didx_v, csem)

    def stg(k, c):
        pltpu.sync_copy(t_hbm.at[cid, pl.ds(r0 + k * SROWS, SROWS)], stage_v)
        pltpu.sync_copy(stage_v, t_sh.at[pl.ds(r0 + k * SROWS, SROWS)])
        return c
    lax.fori_loop(0, NZCP, stg, 0)

    _zero_stage(stage_v, SROWS, TH)

    def zcp(k, c):
        pltpu.sync_copy(stage_v, acc_sh.at[pl.ds(r0 + k * SROWS, SROWS)])
        return c
    lax.fori_loop(0, NZCP, zcp, 0)
    pltpu.make_async_copy(src_hbm.at[sid], sidx_v, csem).wait()
    pltpu.make_async_copy(dst_hbm.at[sid], didx_v, csem).wait()
    plsc.subcore_barrier()

    for b in range(PREF):
        pltpu.async_copy(t_sh.at[sidx_v.at[b]], rows_v.at[b], gsem.at[b])

    def outer(o, carry):
        for j in range(NBUF):
            c = o * NBUF + j
            cp = c + PREF
            bp = (j + PREF) % NBUF

            @pl.when(jnp.logical_and(cp >= NBUF, cp < TCH))
            def _():
                pltpu.make_async_copy(rows_v.at[bp],
                                      acc_sh.at[didx_v.at[0]],
                                      ssem.at[bp]).wait()

            @pl.when(cp < TCH)
            def _():
                pltpu.async_copy(t_sh.at[sidx_v.at[cp]], rows_v.at[bp],
                                 gsem.at[bp])

            pltpu.make_async_copy(t_sh.at[sidx_v.at[c]], rows_v.at[j],
                                  gsem.at[j]).wait()
            pltpu.async_copy(rows_v.at[j], acc_sh.at[didx_v.at[c]],
                             ssem.at[j], add=True)
        return carry
    lax.fori_loop(0, NOUT, outer, 0)

    for j in range(NBUF):
        pltpu.make_async_copy(rows_v.at[j], acc_sh.at[didx_v.at[0]],
                              ssem.at[j]).wait()
    plsc.subcore_barrier()

    def ocp(k, c):
        pltpu.sync_copy(acc_sh.at[pl.ds(r0 + k * SROWS, SROWS)], stage_v)
        pltpu.sync_copy(stage_v, out_hbm.at[cid, pl.ds(r0 + k * SROWS, SROWS)])
        return c
    lax.fori_loop(0, NZCP, ocp, 0)


RB = 1024
_GRID = NP // RB


def _full(shape):
    return pl.BlockSpec(shape, lambda i: tuple(0 for _ in shape))


def _rows(width, leading=None):
    if leading is None:
        return pl.BlockSpec((RB, width), lambda i: (i, 0))
    return pl.BlockSpec((leading, RB, width), lambda i: (0, i, 0))


def _split_t(t_ref, th):
    t_ref[0] = th[:, :TH]
    t_ref[1] = th[:, TH:]


_T_SHAPE = jax.ShapeDtypeStruct((NC, NP, TH), jnp.float32)


def _u1_body(x_ref, w_ref, u_ref):
    u_ref[...] = jnp.dot(x_ref[...], w_ref[...],
                         preferred_element_type=jnp.float32)


def _stage_u1(x_pad, w1):
    return pl.pallas_call(
        _u1_body,
        grid=(_GRID,),
        in_specs=[_rows(IN_DIM), _full((IN_DIM, HID))],
        out_specs=_rows(HID),
        out_shape=jax.ShapeDtypeStruct((NP, HID), jnp.float32),
    )(x_pad, w1)


def _stage0_body(cnt_ref, u_ref, dinv_ref, t_ref):
    d = lax.rsqrt(cnt_ref[:, 0:1] + 1.0)
    dinv_ref[...] = d
    _split_t(t_ref, d * u_ref[...])


def _stage0(cnt, u1):
    return pl.pallas_call(
        _stage0_body,
        grid=(_GRID,),
        in_specs=[_rows(DEGW), _rows(HID)],
        out_specs=[_rows(1), _rows(TH, leading=NC)],
        out_shape=[
            jax.ShapeDtypeStruct((NP, 1), jnp.float32),
            _T_SHAPE,
        ],
    )(cnt, u1)


def _combine(S_ref, uprev, d, b, relu):
    S = jnp.concatenate([S_ref[0], S_ref[1]], axis=1)
    g = d * S + (d * d) * uprev + b
    if relu:
        g = jnp.maximum(g, 0.0)
    return g


def _project_body(S_ref, up_ref, d_ref, b_ref, w_ref, u_ref, t_ref, *, relu):
    g = _combine(S_ref, up_ref[...], d_ref[...], b_ref[...], relu)
    u = jnp.dot(g, w_ref[...], preferred_element_type=jnp.float32)
    u_ref[...] = u
    _split_t(t_ref, d_ref[...] * u)


def _stage_project(S, uprev, dinv, b, w, relu):
    return pl.pallas_call(
        functools.partial(_project_body, relu=relu),
        grid=(_GRID,),
        in_specs=[_rows(TH, leading=NC), _rows(HID), _rows(1),
                  _full((1, HID)), _full((HID, HID))],
        out_specs=[_rows(HID), _rows(TH, leading=NC)],
        out_shape=[
            jax.ShapeDtypeStruct((NP, HID), jnp.float32),
            _T_SHAPE,
        ],
    )(S, uprev, dinv, b.reshape(1, HID), w)


def _project2_body(S_ref, up_ref, d_ref, b_ref, wa_ref, ws_ref,
                   ua_ref, ta_ref, us_ref, ts_ref):
    g = _combine(S_ref, up_ref[...], d_ref[...], b_ref[...], relu=False)
    d = d_ref[...]
    ua = jnp.dot(g, wa_ref[...], preferred_element_type=jnp.float32)
    ua_ref[...] = ua
    _split_t(ta_ref, d * ua)
    us = jnp.dot(g, ws_ref[...], preferred_element_type=jnp.float32)
    us_ref[...] = us
    _split_t(ts_ref, d * us)


def _stage_project2(S, uprev, dinv, b, wa, ws):
    return pl.pallas_call(
        _project2_body,
        grid=(_GRID,),
        in_specs=[_rows(TH, leading=NC), _rows(HID), _rows(1),
                  _full((1, HID)), _full((HID, HID)), _full((HID, HID))],
        out_specs=[_rows(HID), _rows(TH, leading=NC),
                   _rows(HID), _rows(TH, leading=NC)],
        out_shape=[jax.ShapeDtypeStruct((NP, HID), jnp.float32), _T_SHAPE,
                   jax.ShapeDtypeStruct((NP, HID), jnp.float32), _T_SHAPE],
    )(S, uprev, dinv, b.reshape(1, HID), wa, ws)


def _stage_relu_body(S_ref, up_ref, d_ref, b_ref, a_ref, t_ref):
    g = _combine(S_ref, up_ref[...], d_ref[...], b_ref[...], relu=True)
    a_ref[...] = g
    _split_t(t_ref, d_ref[...] * g)


def _stage_relu(S, uprev, dinv, b):
    return pl.pallas_call(
        _stage_relu_body,
        grid=(_GRID,),
        in_specs=[_rows(TH, leading=NC), _rows(HID), _rows(1), _full((1, HID))],
        out_specs=[_rows(HID), _rows(TH, leading=NC)],
        out_shape=[
            jax.ShapeDtypeStruct((NP, HID), jnp.float32),
            _T_SHAPE,
        ],
    )(S, uprev, dinv, b.reshape(1, HID))


def _stage_out_body(S_ref, up_ref, d_ref, b_ref, w_ref, x_ref):
    d = d_ref[...]
    S = jnp.concatenate([S_ref[0], S_ref[1]], axis=1)
    v = d * S + (d * d) * up_ref[...]
    x_ref[...] = jnp.dot(v, w_ref[...], preferred_element_type=jnp.float32) + b_ref[...]


def _stage_out(S, uprev, dinv, b, w):
    return pl.pallas_call(
        _stage_out_body,
        grid=(_GRID,),
        in_specs=[_rows(TH, leading=NC), _rows(HID), _rows(1),
                  _full((1, IN_DIM)), _full((HID, IN_DIM))],
        out_specs=_rows(IN_DIM),
        out_shape=jax.ShapeDtypeStruct((N, IN_DIM), jnp.float32),
    )(S, uprev, dinv, b.reshape(1, IN_DIM), w)


def _stage_hs_body(S_ref, up_ref, d_ref, b_ref, hs_ref):
    hs_ref[...] = _combine(S_ref, up_ref[...], d_ref[...], b_ref[...],
                           relu=False)


def _stage_hs(S, uprev, dinv, b):
    return pl.pallas_call(
        _stage_hs_body,
        grid=(_GRID,),
        in_specs=[_rows(TH, leading=NC), _rows(HID), _rows(1), _full((1, HID))],
        out_specs=_rows(HID),
        out_shape=jax.ShapeDtypeStruct((NP, HID), jnp.float32),
    )(S, uprev, dinv, b.reshape(1, HID))


MMB = 1024


def _struct_body(a_ref, b_ref, o_ref):
    o_ref[...] = lax.dot_general(
        a_ref[...], b_ref[...], (((1,), (1,)), ((), ())),
        preferred_element_type=jnp.float32)


def _struct_decode(hs):
    g = pl.cdiv(N, MMB)
    return pl.pallas_call(
        _struct_body,
        grid=(g, g),
        in_specs=[
            pl.BlockSpec((MMB, HID), lambda i, j: (i, 0)),
            pl.BlockSpec((MMB, HID), lambda i, j: (j, 0)),
        ],
        out_specs=pl.BlockSpec((MMB, MMB), lambda i, j: (i, j)),
        out_shape=jax.ShapeDtypeStruct((N, N), jnp.float32),
        cost_estimate=pl.CostEstimate(
            flops=2 * N * N * HID,
            bytes_accessed=N * N * 4 + 2 * N * HID * 4,
            transcendentals=0),
    )(hs, hs)


def kernel(x, edge_index, enc_W1, enc_b1, enc_W2, enc_b2,
           attr_W1, attr_b1, attr_W2, attr_b2, str_W1, str_b1):
    src = edge_index[0]
    dst = edge_index[1]
    pad = E_PAD - E
    src_p = jnp.concatenate([src, jnp.full((pad,), N, jnp.int32)])
    dst_p = jnp.concatenate([dst, jnp.full((pad,), N, jnp.int32)])
    src_t = src_p.reshape(NS, TCH, CHUNK)
    dst_t = dst_p.reshape(NS, TCH, CHUNK)
    dst_deg = dst_t

    x_pad = jnp.pad(x, ((0, NP - N), (0, 0)))

    u1 = _stage_u1(x_pad, enc_W1)
    cnt = _deg_kernel(dst_deg)
    dinv, t1 = _stage0(cnt, u1)

    S1 = _prop_kernel(t1, src_t, dst_t)
    u2, t2 = _stage_project(S1, u1, dinv, enc_b1, enc_W2, relu=True)

    S2 = _prop_kernel(t2, src_t, dst_t)
    u3, t3, u4, t4 = _stage_project2(S2, u2, dinv, enc_b2, attr_W1, str_W1)

    S4 = _prop_kernel(t4, src_t, dst_t)
    hs = _stage_hs(S4, u4, dinv, str_b1)

    s_ = _struct_decode(hs)
    S3 = _prop_kernel(t3, src_t, dst_t)
    a, t5 = _stage_relu(S3, u3, dinv, attr_b1)
    S5 = _prop_kernel(t5, src_t, dst_t)
    x_ = _stage_out(S5, a, dinv, attr_b2, attr_W2)
    return x_, s_


# speedup vs baseline: 1.1482x; 1.0420x over previous
import functools

import jax
import jax.numpy as jnp
from jax import lax
from jax.experimental import pallas as pl
from jax.experimental.pallas import tpu as pltpu
from jax.experimental.pallas import tpu_sc as plsc

N = 10000
E = 320000
IN_DIM = 128
HID = 64

NC = 2
NS = 16
CHUNK = 128
TCH = 160
TH = HID // 2
EPT = TCH * CHUNK
E_PAD = NS * EPT
NP = 10240
RPT = NP // NS
DEGW = 16
SROWS = 128
NZCP = RPT // SROWS
NBUF = 8
PREF = 4

_mesh = plsc.VectorSubcoreMesh(core_axis_name="c", subcore_axis_name="s",
                               num_cores=1)
_mesh2 = plsc.VectorSubcoreMesh(core_axis_name="c", subcore_axis_name="s",
                                num_cores=2)
_sc_params = pltpu.CompilerParams(use_tc_tiling_on_sc=False)


def _zero_stage(stage_v, nrows, width):
    def body(i, c):
        for j in range(width // 16):
            stage_v[i, pl.ds(j * 16, 16)] = jnp.zeros((16,), jnp.float32)
        return c
    lax.fori_loop(0, nrows, body, 0)


@functools.partial(
    pl.kernel,
    out_type=jax.ShapeDtypeStruct((NP, DEGW), jnp.float32),
    mesh=_mesh,
    scratch_types=[
        pltpu.VMEM((TCH, CHUNK), jnp.int32),
        pltpu.VMEM((CHUNK, DEGW), jnp.float32),
        pltpu.VMEM((SROWS, DEGW), jnp.float32),
        pltpu.VMEM_SHARED((NP, DEGW), jnp.float32),
        pltpu.SemaphoreType.DMA((NBUF,)),
        pltpu.SemaphoreType.DMA,
    ],
    compiler_params=_sc_params,
)
def _deg_kernel(dst_hbm, out_hbm, didx_v, ones_v, stage_v, acc_sh, ssem, sem):
    sid = lax.axis_index("s")
    r0 = sid * RPT

    pltpu.async_copy(dst_hbm.at[sid], didx_v, sem)

    def fill_ones(i, c):
        ones_v[i, pl.ds(0, 16)] = jnp.ones((16,), jnp.float32)
        return c
    lax.fori_loop(0, CHUNK, fill_ones, 0)
    _zero_stage(stage_v, SROWS, DEGW)

    def zcp(k, c):
        pltpu.sync_copy(stage_v, acc_sh.at[pl.ds(r0 + k * SROWS, SROWS)])
        return c
    lax.fori_loop(0, NZCP, zcp, 0)
    pltpu.make_async_copy(dst_hbm.at[sid], didx_v, sem).wait()
    plsc.subcore_barrier()

    def outer(o, carry):
        for j in range(NBUF):
            c = o * NBUF + j

            @pl.when(c >= NBUF)
            def _():
                pltpu.make_async_copy(ones_v, acc_sh.at[didx_v.at[0]],
                                      ssem.at[j]).wait()
            pltpu.async_copy(ones_v, acc_sh.at[didx_v.at[c]], ssem.at[j],
                             add=True)
        return carry
    lax.fori_loop(0, TCH // NBUF, outer, 0)
    for j in range(NBUF):
        pltpu.make_async_copy(ones_v, acc_sh.at[didx_v.at[0]],
                              ssem.at[j]).wait()
    plsc.subcore_barrier()

    def ocp(k, c):
        pltpu.sync_copy(acc_sh.at[pl.ds(r0 + k * SROWS, SROWS)], stage_v)
        pltpu.sync_copy(stage_v, out_hbm.at[pl.ds(r0 + k * SROWS, SROWS)])
        return c
    lax.fori_loop(0, NZCP, ocp, 0)


NOUT = TCH // NBUF


@functools.partial(
    pl.kernel,
    out_type=jax.ShapeDtypeStruct((NC, NP, TH), jnp.float32),
    mesh=_mesh2,
    scratch_types=[
        pltpu.VMEM((TCH, CHUNK), jnp.int32),
        pltpu.VMEM((TCH, CHUNK), jnp.int32),
        pltpu.VMEM((NBUF, CHUNK, TH), jnp.float32),
        pltpu.VMEM((SROWS, TH), jnp.float32),
        pltpu.VMEM_SHARED((NP, TH), jnp.float32),
        pltpu.VMEM_SHARED((NP, TH), jnp.float32),
        pltpu.SemaphoreType.DMA((NBUF,)),
        pltpu.SemaphoreType.DMA((NBUF,)),
        pltpu.SemaphoreType.DMA,
    ],
    compiler_params=_sc_params,
    cost_estimate=pl.CostEstimate(
        flops=E_PAD * TH,
        bytes_accessed=2 * NC * E_PAD * TH * 4,
        transcendentals=0),
)
def _prop_kernel(t_hbm, src_hbm, dst_hbm, out_hbm,
                 sidx_v, didx_v, rows_v, stage_v, t_sh, acc_sh,
                 gsem, ssem, csem):
    cid = lax.axis_index("c")
    sid = lax.axis_index("s")
    r0 = sid * RPT

    pltpu.async_copy(src_hbm.at[sid], sidx_v, csem)
    pltpu.async_copy(dst_hbm.at[sid], didx_v, csem)

    def stg(k, c):
        pltpu.sync_copy(t_hbm.at[cid, pl.ds(r0 + k * SROWS, SROWS)], stage_v)
        pltpu.sync_copy(stage_v, t_sh.at[pl.ds(r0 + k * SROWS, SROWS)])
        return c
    lax.fori_loop(0, NZCP, stg, 0)

    _zero_stage(stage_v, SROWS, TH)

    def zcp(k, c):
        pltpu.sync_copy(stage_v, acc_sh.at[pl.ds(r0 + k * SROWS, SROWS)])
        return c
    lax.fori_loop(0, NZCP, zcp, 0)
    pltpu.make_async_copy(src_hbm.at[sid], sidx_v, csem).wait()
    pltpu.make_async_copy(dst_hbm.at[sid], didx_v, csem).wait()
    plsc.subcore_barrier()

    for b in range(PREF):
        pltpu.async_copy(t_sh.at[sidx_v.at[b]], rows_v.at[b], gsem.at[b])

    def outer(o, carry):
        for j in range(NBUF):
            c = o * NBUF + j
            cp = c + PREF
            bp = (j + PREF) % NBUF

            @pl.when(jnp.logical_and(cp >= NBUF, cp < TCH))
            def _():
                pltpu.make_async_copy(rows_v.at[bp],
                                      acc_sh.at[didx_v.at[0]],
                                      ssem.at[bp]).wait()

            @pl.when(cp < TCH)
            def _():
                pltpu.async_copy(t_sh.at[sidx_v.at[cp]], rows_v.at[bp],
                                 gsem.at[bp])

            pltpu.make_async_copy(t_sh.at[sidx_v.at[c]], rows_v.at[j],
                                  gsem.at[j]).wait()
            pltpu.async_copy(rows_v.at[j], acc_sh.at[didx_v.at[c]],
                             ssem.at[j], add=True)
        return carry
    lax.fori_loop(0, NOUT, outer, 0)

    for j in range(NBUF):
        pltpu.make_async_copy(rows_v.at[j], acc_sh.at[didx_v.at[0]],
                              ssem.at[j]).wait()
    plsc.subcore_barrier()

    def ocp(k, c):
        pltpu.sync_copy(acc_sh.at[pl.ds(r0 + k * SROWS, SROWS)], stage_v)
        pltpu.sync_copy(stage_v, out_hbm.at[cid, pl.ds(r0 + k * SROWS, SROWS)])
        return c
    lax.fori_loop(0, NZCP, ocp, 0)


RB = 1024
_GRID = NP // RB


def _full(shape):
    return pl.BlockSpec(shape, lambda i: tuple(0 for _ in shape))


def _rows(width, leading=None):
    if leading is None:
        return pl.BlockSpec((RB, width), lambda i: (i, 0))
    return pl.BlockSpec((leading, RB, width), lambda i: (0, i, 0))


def _split_t(t_ref, th):
    t_ref[0] = th[:, :TH]
    t_ref[1] = th[:, TH:]


_T_SHAPE = jax.ShapeDtypeStruct((NC, NP, TH), jnp.float32)


def _u1_body(x_ref, w_ref, u_ref):
    u_ref[...] = jnp.dot(x_ref[...], w_ref[...],
                         preferred_element_type=jnp.float32)


def _stage_u1(x_pad, w1):
    return pl.pallas_call(
        _u1_body,
        grid=(_GRID,),
        in_specs=[_rows(IN_DIM), _full((IN_DIM, HID))],
        out_specs=_rows(HID),
        out_shape=jax.ShapeDtypeStruct((NP, HID), jnp.float32),
    )(x_pad, w1)


def _stage0_body(cnt_ref, u_ref, dinv_ref, t_ref):
    d = lax.rsqrt(cnt_ref[:, 0:1] + 1.0)
    dinv_ref[...] = d
    _split_t(t_ref, d * u_ref[...])


def _stage0(cnt, u1):
    return pl.pallas_call(
        _stage0_body,
        grid=(_GRID,),
        in_specs=[_rows(DEGW), _rows(HID)],
        out_specs=[_rows(1), _rows(TH, leading=NC)],
        out_shape=[
            jax.ShapeDtypeStruct((NP, 1), jnp.float32),
            _T_SHAPE,
        ],
    )(cnt, u1)


def _combine(S_ref, uprev, d, b, relu):
    S = jnp.concatenate([S_ref[0], S_ref[1]], axis=1)
    g = d * S + (d * d) * uprev + b
    if relu:
        g = jnp.maximum(g, 0.0)
    return g


def _project_body(S_ref, up_ref, d_ref, b_ref, w_ref, u_ref, t_ref, *, relu):
    g = _combine(S_ref, up_ref[...], d_ref[...], b_ref[...], relu)
    u = jnp.dot(g, w_ref[...], preferred_element_type=jnp.float32)
    u_ref[...] = u
    _split_t(t_ref, d_ref[...] * u)


def _stage_project(S, uprev, dinv, b, w, relu):
    return pl.pallas_call(
        functools.partial(_project_body, relu=relu),
        grid=(_GRID,),
        in_specs=[_rows(TH, leading=NC), _rows(HID), _rows(1),
                  _full((1, HID)), _full((HID, HID))],
        out_specs=[_rows(HID), _rows(TH, leading=NC)],
        out_shape=[
            jax.ShapeDtypeStruct((NP, HID), jnp.float32),
            _T_SHAPE,
        ],
    )(S, uprev, dinv, b.reshape(1, HID), w)


def _project2_body(S_ref, up_ref, d_ref, b_ref, wa_ref, ws_ref,
                   ua_ref, ta_ref, us_ref, ts_ref):
    g = _combine(S_ref, up_ref[...], d_ref[...], b_ref[...], relu=False)
    d = d_ref[...]
    ua = jnp.dot(g, wa_ref[...], preferred_element_type=jnp.float32)
    ua_ref[...] = ua
    _split_t(ta_ref, d * ua)
    us = jnp.dot(g, ws_ref[...], preferred_element_type=jnp.float32)
    us_ref[...] = us
    _split_t(ts_ref, d * us)


def _stage_project2(S, uprev, dinv, b, wa, ws):
    return pl.pallas_call(
        _project2_body,
        grid=(_GRID,),
        in_specs=[_rows(TH, leading=NC), _rows(HID), _rows(1),
                  _full((1, HID)), _full((HID, HID)), _full((HID, HID))],
        out_specs=[_rows(HID), _rows(TH, leading=NC),
                   _rows(HID), _rows(TH, leading=NC)],
        out_shape=[jax.ShapeDtypeStruct((NP, HID), jnp.float32), _T_SHAPE,
                   jax.ShapeDtypeStruct((NP, HID), jnp.float32), _T_SHAPE],
    )(S, uprev, dinv, b.reshape(1, HID), wa, ws)


def _stage_relu_body(S_ref, up_ref, d_ref, b_ref, a_ref, t_ref):
    g = _combine(S_ref, up_ref[...], d_ref[...], b_ref[...], relu=True)
    a_ref[...] = g
    _split_t(t_ref, d_ref[...] * g)


def _stage_relu(S, uprev, dinv, b):
    return pl.pallas_call(
        _stage_relu_body,
        grid=(_GRID,),
        in_specs=[_rows(TH, leading=NC), _rows(HID), _rows(1), _full((1, HID))],
        out_specs=[_rows(HID), _rows(TH, leading=NC)],
        out_shape=[
            jax.ShapeDtypeStruct((NP, HID), jnp.float32),
            _T_SHAPE,
        ],
    )(S, uprev, dinv, b.reshape(1, HID))


def _stage_out_body(S_ref, up_ref, d_ref, b_ref, w_ref, x_ref):
    d = d_ref[...]
    S = jnp.concatenate([S_ref[0], S_ref[1]], axis=1)
    v = d * S + (d * d) * up_ref[...]
    x_ref[...] = jnp.dot(v, w_ref[...], preferred_element_type=jnp.float32) + b_ref[...]


def _stage_out(S, uprev, dinv, b, w):
    return pl.pallas_call(
        _stage_out_body,
        grid=(_GRID,),
        in_specs=[_rows(TH, leading=NC), _rows(HID), _rows(1),
                  _full((1, IN_DIM)), _full((HID, IN_DIM))],
        out_specs=_rows(IN_DIM),
        out_shape=jax.ShapeDtypeStruct((N, IN_DIM), jnp.float32),
    )(S, uprev, dinv, b.reshape(1, IN_DIM), w)


def _stage_hs_body(S_ref, up_ref, d_ref, b_ref, hs_ref):
    hs_ref[...] = _combine(S_ref, up_ref[...], d_ref[...], b_ref[...],
                           relu=False)


def _stage_hs(S, uprev, dinv, b):
    return pl.pallas_call(
        _stage_hs_body,
        grid=(_GRID,),
        in_specs=[_rows(TH, leading=NC), _rows(HID), _rows(1), _full((1, HID))],
        out_specs=_rows(HID),
        out_shape=jax.ShapeDtypeStruct((NP, HID), jnp.float32),
    )(S, uprev, dinv, b.reshape(1, HID))


MMB = 1024


def _struct_body(a_ref, b_ref, o_ref):
    o_ref[...] = lax.dot_general(
        a_ref[...], b_ref[...], (((1,), (1,)), ((), ())),
        preferred_element_type=jnp.float32)


MMC = 2048


def _struct_decode(hs):
    gi = pl.cdiv(N, MMB)
    gj = pl.cdiv(N, MMC)
    return pl.pallas_call(
        _struct_body,
        grid=(gi, gj),
        in_specs=[
            pl.BlockSpec((MMB, HID), lambda i, j: (i, 0)),
            pl.BlockSpec((MMC, HID), lambda i, j: (j, 0)),
        ],
        out_specs=pl.BlockSpec((MMB, MMC), lambda i, j: (i, j)),
        out_shape=jax.ShapeDtypeStruct((N, N), jnp.float32),
        cost_estimate=pl.CostEstimate(
            flops=2 * N * N * HID,
            bytes_accessed=N * N * 4 + 2 * N * HID * 4,
            transcendentals=0),
    )(hs, hs)


def kernel(x, edge_index, enc_W1, enc_b1, enc_W2, enc_b2,
           attr_W1, attr_b1, attr_W2, attr_b2, str_W1, str_b1):
    src = edge_index[0]
    dst = edge_index[1]
    pad = E_PAD - E
    src_p = jnp.concatenate([src, jnp.full((pad,), N, jnp.int32)])
    dst_p = jnp.concatenate([dst, jnp.full((pad,), N, jnp.int32)])
    src_t = src_p.reshape(NS, TCH, CHUNK)
    dst_t = dst_p.reshape(NS, TCH, CHUNK)
    dst_deg = dst_t

    x_pad = jnp.pad(x, ((0, NP - N), (0, 0)))

    u1 = _stage_u1(x_pad, enc_W1)
    cnt = _deg_kernel(dst_deg)
    dinv, t1 = _stage0(cnt, u1)

    S1 = _prop_kernel(t1, src_t, dst_t)
    u2, t2 = _stage_project(S1, u1, dinv, enc_b1, enc_W2, relu=True)

    S2 = _prop_kernel(t2, src_t, dst_t)
    u3, t3, u4, t4 = _stage_project2(S2, u2, dinv, enc_b2, attr_W1, str_W1)

    S4 = _prop_kernel(t4, src_t, dst_t)
    hs = _stage_hs(S4, u4, dinv, str_b1)

    s_ = _struct_decode(hs)
    S3 = _prop_kernel(t3, src_t, dst_t)
    a, t5 = _stage_relu(S3, u3, dinv, attr_b1)
    S5 = _prop_kernel(t5, src_t, dst_t)
    x_ = _stage_out(S5, a, dinv, attr_b2, attr_W2)
    return x_, s_
